# baseline (speedup 1.0000x reference)
import functools
import math

import jax
import jax.numpy as jnp
from jax import lax
from jax.experimental import pallas as pl
from jax.experimental.pallas import tpu as pltpu
from jax.experimental.pallas import tpu_sc as plsc

_VOCAB = 1000000
_D = 64
_DP = 128
_BATCH = 4096
_SEQ = 200
_SCALE = math.sqrt(_D)

_NW = 32
_B_TOT = _BATCH * _SEQ
_PER_W = _B_TOT // _NW
_CHUNK = 128
_NCHUNK = _PER_W // _CHUNK
_NBUF = 5


def _body(x_hbm, tab_hbm, out_hbm, idx_v, rows_v, s0, s1, s2, s3, s4):
    sems = (s0, s1, s2, s3, s4)
    wid = lax.axis_index("s") * 2 + lax.axis_index("c")
    base = wid * _PER_W

    pltpu.sync_copy(x_hbm.at[pl.ds(base, _PER_W)], idx_v)

    def gather_parts(c, b):
        isl = idx_v.at[pl.ds(c * _CHUNK, _CHUNK)]
        return tab_hbm.at[isl], rows_v.at[b], sems[b]

    def start_gather(c, b):
        src, dst, sem = gather_parts(c, b)
        pltpu.async_copy(src, dst, sem)

    def drain_gather(c, b):
        src, dst, sem = gather_parts(c, b)
        pltpu.make_async_copy(src, dst, sem).wait()

    def write(c, b):
        pltpu.sync_copy(rows_v.at[b],
                        out_hbm.at[pl.ds(base + c * _CHUNK, _CHUNK)])

    for c in range(_NBUF - 1):
        start_gather(c, c)

    def group_fn(k, _):
        for u in range(_NBUF):
            c = k * _NBUF + u
            drain_gather(c, u)

            @pl.when(c + _NBUF - 1 < _NCHUNK)
            def _():
                start_gather(c + _NBUF - 1, (u + _NBUF - 1) % _NBUF)

            write(c, u)
        return 0

    lax.fori_loop(0, _NCHUNK // _NBUF, group_fn, 0)


_emb = functools.partial(
    pl.kernel,
    mesh=plsc.VectorSubcoreMesh(core_axis_name="c", subcore_axis_name="s"),
    out_type=jax.ShapeDtypeStruct((_B_TOT, _DP), jnp.float32),
    scratch_types=[
        pltpu.VMEM((_PER_W,), jnp.int32),
        pltpu.VMEM((_NBUF, _CHUNK, _DP), jnp.float32),
        pltpu.SemaphoreType.DMA,
        pltpu.SemaphoreType.DMA,
        pltpu.SemaphoreType.DMA,
        pltpu.SemaphoreType.DMA,
        pltpu.SemaphoreType.DMA,
    ],
)(_body)


@jax.jit
def kernel(x, table):
    xf = x.reshape(_B_TOT)
    table_p = jnp.pad(table, ((0, 0), (0, _DP - _D))) * _SCALE
    out = _emb(xf, table_p)
    return out[:, :_D].reshape(_BATCH, _SEQ, _D)
